# Initial kernel scaffold; baseline (speedup 1.0000x reference)
import jax
import jax.numpy as jnp
from jax.experimental import pallas as pl


def kernel(x, weight, bias):
    raise NotImplementedError("write your pallas kernel here")


# speedup vs baseline: 2.1269x; 2.1269x over previous
import functools

import jax
import jax.numpy as jnp
from jax.experimental import pallas as pl
from jax.experimental.pallas import tpu as pltpu

_GELU_C0 = 0.044715
_GELU_C1 = 0.7978845608028654


def _gelu_tanh(v):
    return (v * 0.5) * (1.0 + jnp.tanh(_GELU_C1 * (v + _GELU_C0 * v * v * v)))


def _fused_kernel(x_ref, w_ref, b_ref, o_ref, g_s, *, d_size, cout):
    c4 = w_ref.shape[0]
    dhw = x_ref.shape[2]
    hw = dhw // d_size
    n_k = 4 * hw // 128
    do = 2 * d_size - 1
    row_stride = do * n_k

    v = jnp.dot(w_ref[...], x_ref[0], preferred_element_type=jnp.float32)
    v = v + jnp.broadcast_to(b_ref[...], (c4, dhw))
    g_s[...] = _gelu_tanh(v)

    fill = jnp.broadcast_to(_gelu_tanh(b_ref[0:cout]), (cout, 128))

    il = jax.lax.broadcasted_iota(jnp.int32, (c4, 128), 1)
    kh_sel = ((il >> 5) & 1) == 0
    kw_sel = (il & 1) == 0
    base_idx = 16 * ((il >> 6) & 1) + ((il >> 1) & 15)
    idx_q = [base_idx + 32 * q for q in range(4)]

    for d in range(d_size):
        for k in range(n_k):
            p, q = k // 4, k % 4
            src = g_s[:, hw * d + 128 * p:hw * d + 128 * (p + 1)]
            gall = jnp.take_along_axis(src, idx_q[q], axis=1)
            piece = jnp.where(
                kh_sel[0:cout],
                jnp.where(kw_sel[0:cout], gall[0:cout], gall[cout:2 * cout]),
                jnp.where(kw_sel[0:cout], gall[2 * cout:3 * cout], gall[3 * cout:]),
            )
            o_ref[0, pl.ds(2 * d * n_k + k, cout, row_stride), :] = piece
        if d + 1 < d_size:
            for k in range(n_k):
                o_ref[0, pl.ds((2 * d + 1) * n_k + k, cout, row_stride), :] = fill


def kernel(x, weight, bias):
    n, cin, d_size, h, w = x.shape
    cout = weight.shape[1]
    do, ho, wo = 2 * d_size - 1, 2 * h, 2 * w
    dhw = d_size * h * w
    c4 = 4 * cout

    xr = x.reshape(n, cin, dhw)
    w4 = jnp.transpose(weight[:, :, 0, :, :], (2, 3, 1, 0)).reshape(c4, cin)
    b4 = jnp.tile(bias, 4).reshape(c4, 1)

    n_rows = cout * do * (ho * wo // 128)
    out = pl.pallas_call(
        functools.partial(_fused_kernel, d_size=d_size, cout=cout),
        out_shape=jax.ShapeDtypeStruct((n, n_rows, 128), jnp.float32),
        grid=(n,),
        in_specs=[
            pl.BlockSpec((1, cin, dhw), lambda i: (i, 0, 0)),
            pl.BlockSpec((c4, cin), lambda i: (0, 0)),
            pl.BlockSpec((c4, 1), lambda i: (0, 0)),
        ],
        out_specs=pl.BlockSpec((1, n_rows, 128), lambda i: (i, 0, 0)),
        scratch_shapes=[pltpu.VMEM((c4, dhw), jnp.float32)],
        compiler_params=pltpu.CompilerParams(
            dimension_semantics=("parallel",)),
    )(xr, w4, b4)
    return out.reshape(n, cout, do, ho, wo)


# speedup vs baseline: 13.5564x; 6.3737x over previous
import functools

import jax
import jax.numpy as jnp
from jax.experimental import pallas as pl
from jax.experimental.pallas import tpu as pltpu

_GELU_C0 = 0.044715
_GELU_C1 = 0.7978845608028654


def _gelu_tanh(v):
    return (v * 0.5) * (1.0 + jnp.tanh(_GELU_C1 * (v + _GELU_C0 * v * v * v)))


def _fused_kernel(x_ref, w_ref, b_ref, o_ref, g_s, *, d_size, h, w, cout):
    dhw = x_ref.shape[1]
    c4 = w_ref.shape[1]

    v = jnp.dot(x_ref[0], w_ref[...], preferred_element_type=jnp.float32)
    v = v + jnp.broadcast_to(b_ref[...], (dhw, c4))
    g_s[...] = _gelu_tanh(v)

    for kh in range(2):
        for kw in range(2):
            t = 2 * kh + kw
            val = g_s[:, t * cout:(t + 1) * cout].reshape(d_size, h, w, cout)
            o_ref[0, pl.ds(0, d_size, 2), pl.ds(kh, h, 2), pl.ds(kw, w, 2), :] = val

    fill = jnp.broadcast_to(
        _gelu_tanh(b_ref[0:1, 0:cout]).reshape(1, 1, cout), (2 * h, 2 * w, cout))
    for d in range(d_size - 1):
        o_ref[0, 2 * d + 1] = fill


def kernel(x, weight, bias):
    n, cin, d_size, h, w = x.shape
    cout = weight.shape[1]
    do, ho, wo = 2 * d_size - 1, 2 * h, 2 * w
    dhw = d_size * h * w
    c4 = 4 * cout

    xt = jnp.transpose(x, (0, 2, 3, 4, 1)).reshape(n, dhw, cin)
    w4t = jnp.transpose(weight[:, :, 0, :, :], (0, 2, 3, 1)).reshape(cin, c4)
    b4 = jnp.tile(bias, 4).reshape(1, c4)

    out5 = pl.pallas_call(
        functools.partial(_fused_kernel, d_size=d_size, h=h, w=w, cout=cout),
        out_shape=jax.ShapeDtypeStruct((n, do, ho, wo, cout), jnp.float32),
        grid=(n,),
        in_specs=[
            pl.BlockSpec((1, dhw, cin), lambda i: (i, 0, 0)),
            pl.BlockSpec((cin, c4), lambda i: (0, 0)),
            pl.BlockSpec((1, c4), lambda i: (0, 0)),
        ],
        out_specs=pl.BlockSpec((1, do, ho, wo, cout), lambda i: (i, 0, 0, 0, 0)),
        scratch_shapes=[pltpu.VMEM((dhw, c4), jnp.float32)],
        compiler_params=pltpu.CompilerParams(
            dimension_semantics=("parallel",)),
    )(xt, w4t, b4)
    return jnp.transpose(out5, (0, 4, 1, 2, 3))
